# baseline (speedup 1.0000x reference)
import jax
import jax.numpy as jnp
from jax import lax
from jax.experimental import pallas as pl
from jax.experimental.pallas import tpu as pltpu

_K = 8192
_DIM = 64
_B = 8192
_TAU = 1.0
_ALPHA = 1.0
_BETA = 1.0
_BR = 1024
_KC = 128
_LN2 = 0.6931471805599453
_LOG2E = 1.4426950408889634
_A = _TAU * _LOG2E
_A2 = _A * _A


def _body(x_ref, p_ref, out_ref, acc_ref, l_ref):
    i = pl.program_id(0)
    nsteps = pl.num_programs(0)
    x = x_ref[...]

    @pl.when(i == 0)
    def _init():
        p = p_ref[...]
        onesA = jnp.full((1, _DIM), _A2, dtype=jnp.float32)
        p2 = lax.dot_general(onesA, p * p, (((1,), (1,)), ((), ())),
                             preferred_element_type=jnp.float32)
        onescol = jnp.ones((_K, 1), dtype=jnp.float32)
        l_ref[...] = jnp.concatenate(
            [p, onescol, p2.T], axis=1).astype(jnp.bfloat16)
        acc_ref[0] = 0.0
        acc_ref[1] = 0.0

    xm = (-2.0 * _A2) * x
    x2 = _A2 * jnp.sum(x * x, axis=1, keepdims=True)
    onesc = jnp.ones((_BR, 1), dtype=jnp.float32)
    r_aug = jnp.concatenate(
        [xm, x2, onesc], axis=1).astype(jnp.bfloat16)

    s = jnp.zeros((1, _BR), dtype=jnp.float32)
    umax = jnp.zeros((1, _BR), dtype=jnp.float32)
    for c in range(_K // _KC):
        lc = l_ref[pl.ds(c * _KC, _KC), :]
        t = lax.dot_general(lc, r_aug, (((1,), (1,)), ((), ())),
                            preferred_element_type=jnp.float32)
        t = jnp.maximum(t, 1e-30)
        u = jnp.exp2(-(t * lax.rsqrt(t)))
        s = s + jnp.sum(u, axis=0, keepdims=True)
        umax = jnp.maximum(umax, jnp.max(u, axis=0, keepdims=True))

    mp = -jnp.log(umax) * _LOG2E
    acc_ref[0] += jnp.sum(jnp.log(s) - jnp.log(umax))
    acc_ref[1] += jnp.sum(mp * mp)

    @pl.when(i == nsteps - 1)
    def _finish():
        p = p_ref[...]
        n2 = jnp.sum(p * p, axis=1, keepdims=True)
        pn = p * lax.rsqrt(jnp.maximum(n2, 1e-24))
        c = lax.dot_general(pn, pn, (((0,), (0,)), ((), ())),
                            preferred_element_type=jnp.float32)
        tr = jnp.sum(n2 / jnp.maximum(n2, 1e-24))
        compact = jnp.sum(c * c) - 2.0 * tr + float(_K)
        total = (acc_ref[0] / _B
                 + _ALPHA * (acc_ref[1] / (_A2 * _B * _DIM))
                 + _BETA * compact)
        out_ref[...] = jnp.reshape(total, (1, 1))


def kernel(x_feat, prototypes):
    out = pl.pallas_call(
        _body,
        grid=(_B // _BR,),
        in_specs=[
            pl.BlockSpec((_BR, _DIM), lambda i: (i, 0)),
            pl.BlockSpec((_K, _DIM), lambda i: (0, 0)),
        ],
        out_specs=pl.BlockSpec((1, 1), lambda i: (0, 0)),
        out_shape=jax.ShapeDtypeStruct((1, 1), jnp.float32),
        scratch_shapes=[pltpu.SMEM((2,), jnp.float32),
                        pltpu.VMEM((_K, _DIM + 2), jnp.bfloat16)],
    )(x_feat, prototypes)
    return out[0, 0]


# speedup vs baseline: 1.1003x; 1.1003x over previous
import jax
import jax.numpy as jnp
from jax import lax
from jax.experimental import pallas as pl
from jax.experimental.pallas import tpu as pltpu

_K = 8192
_DIM = 64
_B = 8192
_TAU = 1.0
_ALPHA = 1.0
_BETA = 1.0
_BR = 512
_KC = 256
_LN2 = 0.6931471805599453
_LOG2E = 1.4426950408889634
_A = _TAU * _LOG2E
_A2 = _A * _A


def _body(x_ref, p_ref, out_ref, acc_ref, l_ref):
    i = pl.program_id(0)
    nsteps = pl.num_programs(0)
    x = x_ref[...]

    @pl.when(i == 0)
    def _init():
        p = p_ref[...]
        onesA = jnp.full((1, _DIM), _A2, dtype=jnp.float32)
        p2 = lax.dot_general(onesA, p * p, (((1,), (1,)), ((), ())),
                             preferred_element_type=jnp.float32)
        onescol = jnp.ones((_K, 1), dtype=jnp.float32)
        l_ref[...] = jnp.concatenate(
            [p, onescol, p2.T], axis=1).astype(jnp.bfloat16)
        acc_ref[0] = 0.0
        acc_ref[1] = 0.0

    xm = (-2.0 * _A2) * x
    x2 = _A2 * jnp.sum(x * x, axis=1, keepdims=True)
    onesc = jnp.ones((_BR, 1), dtype=jnp.float32)
    r_aug = jnp.concatenate(
        [xm, x2, onesc], axis=1).astype(jnp.bfloat16)

    s = jnp.zeros((1, _BR), dtype=jnp.float32)
    umax = jnp.zeros((1, _BR), dtype=jnp.float32)
    for c in range(_K // _KC):
        lc = l_ref[pl.ds(c * _KC, _KC), :]
        t = lax.dot_general(lc, r_aug, (((1,), (1,)), ((), ())),
                            preferred_element_type=jnp.float32)
        t = jnp.maximum(t, 1e-30)
        u = jnp.exp2(-(t * lax.rsqrt(t)))
        s = s + jnp.sum(u, axis=0, keepdims=True)
        umax = jnp.maximum(umax, jnp.max(u, axis=0, keepdims=True))

    mp = -jnp.log(umax) * _LOG2E
    acc_ref[0] += jnp.sum(jnp.log(s) - jnp.log(umax))
    acc_ref[1] += jnp.sum(mp * mp)

    @pl.when(i == nsteps - 1)
    def _finish():
        p = p_ref[...]
        n2 = jnp.sum(p * p, axis=1, keepdims=True)
        pn = p * lax.rsqrt(jnp.maximum(n2, 1e-24))
        c = lax.dot_general(pn, pn, (((0,), (0,)), ((), ())),
                            preferred_element_type=jnp.float32)
        tr = jnp.sum(n2 / jnp.maximum(n2, 1e-24))
        compact = jnp.sum(c * c) - 2.0 * tr + float(_K)
        total = (acc_ref[0] / _B
                 + _ALPHA * (acc_ref[1] / (_A2 * _B * _DIM))
                 + _BETA * compact)
        out_ref[...] = jnp.reshape(total, (1, 1))


def kernel(x_feat, prototypes):
    out = pl.pallas_call(
        _body,
        grid=(_B // _BR,),
        in_specs=[
            pl.BlockSpec((_BR, _DIM), lambda i: (i, 0)),
            pl.BlockSpec((_K, _DIM), lambda i: (0, 0)),
        ],
        out_specs=pl.BlockSpec((1, 1), lambda i: (0, 0)),
        out_shape=jax.ShapeDtypeStruct((1, 1), jnp.float32),
        scratch_shapes=[pltpu.SMEM((2,), jnp.float32),
                        pltpu.VMEM((_K, _DIM + 2), jnp.bfloat16)],
    )(x_feat, prototypes)
    return out[0, 0]


# speedup vs baseline: 1.1212x; 1.0189x over previous
import jax
import jax.numpy as jnp
from jax import lax
from jax.experimental import pallas as pl
from jax.experimental.pallas import tpu as pltpu

_K = 8192
_DIM = 64
_B = 8192
_TAU = 1.0
_ALPHA = 1.0
_BETA = 1.0
_BR = 512
_KC = 512
_LN2 = 0.6931471805599453
_LOG2E = 1.4426950408889634
_A = _TAU * _LOG2E
_A2 = _A * _A


def _body(x_ref, p_ref, out_ref, acc_ref, l_ref):
    i = pl.program_id(0)
    nsteps = pl.num_programs(0)
    x = x_ref[...]

    @pl.when(i == 0)
    def _init():
        p = p_ref[...]
        onesA = jnp.full((1, _DIM), _A2, dtype=jnp.float32)
        p2 = lax.dot_general(onesA, p * p, (((1,), (1,)), ((), ())),
                             preferred_element_type=jnp.float32)
        onescol = jnp.ones((_K, 1), dtype=jnp.float32)
        l_ref[...] = jnp.concatenate(
            [p, onescol, p2.T], axis=1).astype(jnp.bfloat16)
        acc_ref[0] = 0.0
        acc_ref[1] = 0.0

    xm = (-2.0 * _A2) * x
    x2 = _A2 * jnp.sum(x * x, axis=1, keepdims=True)
    onesc = jnp.ones((_BR, 1), dtype=jnp.float32)
    r_aug = jnp.concatenate(
        [xm, x2, onesc], axis=1).astype(jnp.bfloat16)

    s = jnp.zeros((1, _BR), dtype=jnp.float32)
    umax = jnp.zeros((1, _BR), dtype=jnp.float32)
    for c in range(_K // _KC):
        lc = l_ref[pl.ds(c * _KC, _KC), :]
        t = lax.dot_general(lc, r_aug, (((1,), (1,)), ((), ())),
                            preferred_element_type=jnp.float32)
        t = jnp.maximum(t, 1e-30)
        u = jnp.exp2(-(t * lax.rsqrt(t)))
        s = s + jnp.sum(u, axis=0, keepdims=True)
        umax = jnp.maximum(umax, jnp.max(u, axis=0, keepdims=True))

    mp = -jnp.log(umax) * _LOG2E
    acc_ref[0] += jnp.sum(jnp.log(s) - jnp.log(umax))
    acc_ref[1] += jnp.sum(mp * mp)

    @pl.when(i == nsteps - 1)
    def _finish():
        p = p_ref[...]
        n2 = jnp.sum(p * p, axis=1, keepdims=True)
        pn = p * lax.rsqrt(jnp.maximum(n2, 1e-24))
        c = lax.dot_general(pn, pn, (((0,), (0,)), ((), ())),
                            preferred_element_type=jnp.float32)
        tr = jnp.sum(n2 / jnp.maximum(n2, 1e-24))
        compact = jnp.sum(c * c) - 2.0 * tr + float(_K)
        total = (acc_ref[0] / _B
                 + _ALPHA * (acc_ref[1] / (_A2 * _B * _DIM))
                 + _BETA * compact)
        out_ref[...] = jnp.reshape(total, (1, 1))


def kernel(x_feat, prototypes):
    out = pl.pallas_call(
        _body,
        grid=(_B // _BR,),
        in_specs=[
            pl.BlockSpec((_BR, _DIM), lambda i: (i, 0)),
            pl.BlockSpec((_K, _DIM), lambda i: (0, 0)),
        ],
        out_specs=pl.BlockSpec((1, 1), lambda i: (0, 0)),
        out_shape=jax.ShapeDtypeStruct((1, 1), jnp.float32),
        scratch_shapes=[pltpu.SMEM((2,), jnp.float32),
                        pltpu.VMEM((_K, _DIM + 2), jnp.bfloat16)],
    )(x_feat, prototypes)
    return out[0, 0]


# speedup vs baseline: 1.1363x; 1.0135x over previous
import jax
import jax.numpy as jnp
from jax import lax
from jax.experimental import pallas as pl
from jax.experimental.pallas import tpu as pltpu

_K = 8192
_DIM = 64
_B = 8192
_TAU = 1.0
_ALPHA = 1.0
_BETA = 1.0
_BR = 512
_KC = 1024
_LN2 = 0.6931471805599453
_LOG2E = 1.4426950408889634
_A = _TAU * _LOG2E
_A2 = _A * _A


def _body(x_ref, p_ref, out_ref, acc_ref, l_ref):
    i = pl.program_id(0)
    nsteps = pl.num_programs(0)
    x = x_ref[...]

    @pl.when(i == 0)
    def _init():
        p = p_ref[...]
        onesA = jnp.full((1, _DIM), _A2, dtype=jnp.float32)
        p2 = lax.dot_general(onesA, p * p, (((1,), (1,)), ((), ())),
                             preferred_element_type=jnp.float32)
        onescol = jnp.ones((_K, 1), dtype=jnp.float32)
        l_ref[...] = jnp.concatenate(
            [p, onescol, p2.T], axis=1).astype(jnp.bfloat16)
        acc_ref[0] = 0.0
        acc_ref[1] = 0.0

    xm = (-2.0 * _A2) * x
    x2 = _A2 * jnp.sum(x * x, axis=1, keepdims=True)
    onesc = jnp.ones((_BR, 1), dtype=jnp.float32)
    r_aug = jnp.concatenate(
        [xm, x2, onesc], axis=1).astype(jnp.bfloat16)

    s = jnp.zeros((1, _BR), dtype=jnp.float32)
    umax = jnp.zeros((1, _BR), dtype=jnp.float32)
    for c in range(_K // _KC):
        lc = l_ref[pl.ds(c * _KC, _KC), :]
        t = lax.dot_general(lc, r_aug, (((1,), (1,)), ((), ())),
                            preferred_element_type=jnp.float32)
        t = jnp.maximum(t, 1e-30)
        u = jnp.exp2(-(t * lax.rsqrt(t)))
        s = s + jnp.sum(u, axis=0, keepdims=True)
        umax = jnp.maximum(umax, jnp.max(u, axis=0, keepdims=True))

    mp = -jnp.log(umax) * _LOG2E
    acc_ref[0] += jnp.sum(jnp.log(s) - jnp.log(umax))
    acc_ref[1] += jnp.sum(mp * mp)

    @pl.when(i == nsteps - 1)
    def _finish():
        p = p_ref[...]
        n2 = jnp.sum(p * p, axis=1, keepdims=True)
        pn = p * lax.rsqrt(jnp.maximum(n2, 1e-24))
        c = lax.dot_general(pn, pn, (((0,), (0,)), ((), ())),
                            preferred_element_type=jnp.float32)
        tr = jnp.sum(n2 / jnp.maximum(n2, 1e-24))
        compact = jnp.sum(c * c) - 2.0 * tr + float(_K)
        total = (acc_ref[0] / _B
                 + _ALPHA * (acc_ref[1] / (_A2 * _B * _DIM))
                 + _BETA * compact)
        out_ref[...] = jnp.reshape(total, (1, 1))


def kernel(x_feat, prototypes):
    out = pl.pallas_call(
        _body,
        grid=(_B // _BR,),
        in_specs=[
            pl.BlockSpec((_BR, _DIM), lambda i: (i, 0)),
            pl.BlockSpec((_K, _DIM), lambda i: (0, 0)),
        ],
        out_specs=pl.BlockSpec((1, 1), lambda i: (0, 0)),
        out_shape=jax.ShapeDtypeStruct((1, 1), jnp.float32),
        scratch_shapes=[pltpu.SMEM((2,), jnp.float32),
                        pltpu.VMEM((_K, _DIM + 2), jnp.bfloat16)],
    )(x_feat, prototypes)
    return out[0, 0]


# speedup vs baseline: 1.1367x; 1.0003x over previous
import jax
import jax.numpy as jnp
from jax import lax
from jax.experimental import pallas as pl
from jax.experimental.pallas import tpu as pltpu

_K = 8192
_DIM = 64
_B = 8192
_TAU = 1.0
_ALPHA = 1.0
_BETA = 1.0
_BR = 512
_KC = 2048
_LN2 = 0.6931471805599453
_LOG2E = 1.4426950408889634
_A = _TAU * _LOG2E
_A2 = _A * _A


def _body(x_ref, p_ref, out_ref, acc_ref, l_ref):
    i = pl.program_id(0)
    nsteps = pl.num_programs(0)
    x = x_ref[...]

    @pl.when(i == 0)
    def _init():
        p = p_ref[...]
        onesA = jnp.full((1, _DIM), _A2, dtype=jnp.float32)
        p2 = lax.dot_general(onesA, p * p, (((1,), (1,)), ((), ())),
                             preferred_element_type=jnp.float32)
        onescol = jnp.ones((_K, 1), dtype=jnp.float32)
        l_ref[...] = jnp.concatenate(
            [p, onescol, p2.T], axis=1).astype(jnp.bfloat16)
        acc_ref[0] = 0.0
        acc_ref[1] = 0.0

    xm = (-2.0 * _A2) * x
    x2 = _A2 * jnp.sum(x * x, axis=1, keepdims=True)
    onesc = jnp.ones((_BR, 1), dtype=jnp.float32)
    r_aug = jnp.concatenate(
        [xm, x2, onesc], axis=1).astype(jnp.bfloat16)

    s = jnp.zeros((1, _BR), dtype=jnp.float32)
    umax = jnp.zeros((1, _BR), dtype=jnp.float32)
    for c in range(_K // _KC):
        lc = l_ref[pl.ds(c * _KC, _KC), :]
        t = lax.dot_general(lc, r_aug, (((1,), (1,)), ((), ())),
                            preferred_element_type=jnp.float32)
        t = jnp.maximum(t, 1e-30)
        u = jnp.exp2(-(t * lax.rsqrt(t)))
        s = s + jnp.sum(u, axis=0, keepdims=True)
        umax = jnp.maximum(umax, jnp.max(u, axis=0, keepdims=True))

    mp = -jnp.log(umax) * _LOG2E
    acc_ref[0] += jnp.sum(jnp.log(s) - jnp.log(umax))
    acc_ref[1] += jnp.sum(mp * mp)

    @pl.when(i == nsteps - 1)
    def _finish():
        p = p_ref[...]
        n2 = jnp.sum(p * p, axis=1, keepdims=True)
        pn = p * lax.rsqrt(jnp.maximum(n2, 1e-24))
        c = lax.dot_general(pn, pn, (((0,), (0,)), ((), ())),
                            preferred_element_type=jnp.float32)
        tr = jnp.sum(n2 / jnp.maximum(n2, 1e-24))
        compact = jnp.sum(c * c) - 2.0 * tr + float(_K)
        total = (acc_ref[0] / _B
                 + _ALPHA * (acc_ref[1] / (_A2 * _B * _DIM))
                 + _BETA * compact)
        out_ref[...] = jnp.reshape(total, (1, 1))


def kernel(x_feat, prototypes):
    out = pl.pallas_call(
        _body,
        grid=(_B // _BR,),
        in_specs=[
            pl.BlockSpec((_BR, _DIM), lambda i: (i, 0)),
            pl.BlockSpec((_K, _DIM), lambda i: (0, 0)),
        ],
        out_specs=pl.BlockSpec((1, 1), lambda i: (0, 0)),
        out_shape=jax.ShapeDtypeStruct((1, 1), jnp.float32),
        scratch_shapes=[pltpu.SMEM((2,), jnp.float32),
                        pltpu.VMEM((_K, _DIM + 2), jnp.bfloat16)],
    )(x_feat, prototypes)
    return out[0, 0]
